# baseline (speedup 1.0000x reference)
import jax
import jax.numpy as jnp
from jax.experimental import pallas as pl

N = 207
B = 16
L = 8
HORIZON = 8
U = 64
NUM_MAT = 5
DIN_D = 80
XP = 16
F32 = jnp.float32
BF16 = jnp.bfloat16

_DNT = (((0,), (0,)), ((), ()))


def _fwd_kernel(x_row_ref, x_nd_ref, s0t_ref, s1t_ref,
                w0x_e_ref, w0h_ru_e_ref, w4_ru_e_ref, b_ru_e_ref,
                w0h_c_e_ref, w4_c_e_ref, b_c_e_ref,
                w0_ru_d_ref, w4_ru_d_ref, b_ru_d_ref,
                w0_c_d_ref, w4_c_d_ref, b_c_d_ref,
                w_projt_ref, b_proj_ref,
                out_ref):
    s0t = s0t_ref[...]
    s1t = s1t_ref[...]

    def dotT(a, w):
        return jax.lax.dot_general(a, w, _DNT, preferred_element_type=F32)

    def diffuse(pieces, st):
        z1 = jnp.concatenate(
            [dotT(p, st).astype(BF16) for p in pieces], axis=0) \
            if len(pieces) > 1 else dotT(pieces[0], st).astype(BF16)
        z2 = jnp.dot(z1, st, preferred_element_type=F32).astype(BF16)
        return z1, z2

    w0x_e = w0x_e_ref[...]
    w0h_ru_e = w0h_ru_e_ref[...]
    w4_ru_e = w4_ru_e_ref[...]
    b_ru_e = b_ru_e_ref[...]
    w0h_c_e = w0h_c_e_ref[...]
    w4_c_e = w4_c_e_ref[...]
    b_c_e = b_c_e_ref[...]

    def ggate(state, zx, w0h, w4, b, g0x):
        sb = state.astype(BF16)
        s3 = sb.reshape(B, N, U)
        zh1a, zh2a = diffuse([s3[i] for i in range(B)], s0t)
        zh1b, zh2b = diffuse([s3[i] for i in range(B)], s1t)
        zx1a, zx2a, zx1b, zx2b = zx
        gates = []
        for i in range(B):
            hs = slice(i * U, (i + 1) * U)
            xs = slice(i * XP, (i + 1) * XP)
            zcat = jnp.concatenate(
                [zh1a[hs], zh2a[hs], zh1b[hs], zh2b[hs],
                 zx1a[xs], zx2a[xs], zx1b[xs], zx2b[xs]], axis=0)
            gates.append(dotT(zcat, w4))
        return (jnp.concatenate(gates, axis=0)
                + jnp.dot(sb, w0h, preferred_element_type=F32) + g0x + b)

    def enc_body(t, h):
        x_row = x_row_ref[pl.ds(t, 1)].reshape(B * N, L)
        x_nd = x_nd_ref[pl.ds(t, 1)].reshape(N, B * XP)
        zx1a, zx2a = diffuse([x_nd], s0t)
        zx1b, zx2b = diffuse([x_nd], s1t)
        zx = (zx1a, zx2a, zx1b, zx2b)
        g0x = jnp.dot(x_row, w0x_e, preferred_element_type=F32)
        ru_raw = ggate(h, zx, w0h_ru_e, w4_ru_e, b_ru_e, g0x[:, :256])
        r = jax.nn.sigmoid(ru_raw[:, :U])
        u = jax.nn.sigmoid(ru_raw[:, 128:128 + U])
        c = jnp.tanh(ggate(r * h, zx, w0h_c_e, w4_c_e, b_c_e, g0x[:, 256:]))
        return u * h + (1.0 - u) * c

    h = jax.lax.fori_loop(0, L, enc_body, jnp.zeros((B * N, U), F32))

    w0_ru_d = w0_ru_d_ref[...]
    w4_ru_d = w4_ru_d_ref[...]
    b_ru_d = b_ru_d_ref[...]
    w0_c_d = w0_c_d_ref[...]
    w4_c_d = w4_c_d_ref[...]
    b_c_d = b_c_d_ref[...]
    w_projt = w_projt_ref[...]
    b_proj = b_proj_ref[0, 0]

    def gconv_d(cat, w0, w4, b):
        cat3 = cat.reshape(B, N, DIN_D)
        z1a, z2a = diffuse([cat3[i] for i in range(B)], s0t)
        z1b, z2b = diffuse([cat3[i] for i in range(B)], s1t)
        gates = []
        for i in range(B):
            sl = slice(i * DIN_D, (i + 1) * DIN_D)
            zcat = jnp.concatenate([z1a[sl], z2a[sl], z1b[sl], z2b[sl]], axis=0)
            gates.append(dotT(zcat, w4))
        return (jnp.concatenate(gates, axis=0)
                + jnp.dot(cat, w0, preferred_element_type=F32) + b)

    def dec_body(t, h):
        projt = jnp.dot(w_projt, h.T, preferred_element_type=F32) + b_proj
        xin = jnp.where(t == 0, jnp.zeros_like(projt), projt).T
        xin16 = jnp.pad(xin, ((0, 0), (0, XP - 1)))
        cat = jnp.concatenate([h, xin16], axis=1).astype(BF16)
        ru_raw = gconv_d(cat, w0_ru_d, w4_ru_d, b_ru_d)
        r = jax.nn.sigmoid(ru_raw[:, :U])
        u = jax.nn.sigmoid(ru_raw[:, 128:128 + U])
        cat2 = jnp.concatenate([r * h, xin16], axis=1).astype(BF16)
        c = jnp.tanh(gconv_d(cat2, w0_c_d, w4_c_d, b_c_d))
        h2 = u * h + (1.0 - u) * c
        proj2 = jnp.dot(w_projt, h2.T, preferred_element_type=F32) + b_proj
        out_ref[pl.ds(t, 1)] = proj2
        return h2

    jax.lax.fori_loop(0, HORIZON, dec_body, h)


def _expand_ru(w):
    z = jnp.zeros(w.shape[:-1] + (U,), w.dtype)
    return jnp.concatenate([w[..., :U], z, w[..., U:], z], axis=-1)


def _split_w(w, dx, dout):
    din = dx + U
    wm = w.reshape(din, NUM_MAT, dout)
    wx, wh = wm[:dx], wm[dx:]
    w0x = wx[:, 0] - wx[:, 2] - wx[:, 4]
    w0h = wh[:, 0] - wh[:, 2] - wh[:, 4]
    pad = jnp.zeros((XP - dx, dout), F32)
    w4 = jnp.concatenate(
        [wh[:, 1], 2.0 * wh[:, 2], wh[:, 3], 2.0 * wh[:, 4],
         wx[:, 1], pad, 2.0 * wx[:, 2], pad, wx[:, 3], pad,
         2.0 * wx[:, 4], pad], axis=0)
    return w0x, w0h, w4


def kernel(inputs, support0, support1, W_ru_e, b_ru_e, W_c_e, b_c_e,
           W_ru_d, b_ru_d, W_c_d, b_c_d, W_proj, b_proj):
    x_row = inputs.reshape(L, B * N, L).astype(BF16)
    x_nd = jnp.pad(inputs.reshape(L, B, N, L).transpose(0, 2, 1, 3),
                   ((0, 0), (0, 0), (0, 0), (0, XP - L))
                   ).reshape(L, N, B * XP).astype(BF16)

    w0x_ru_e, w0h_ru_e, w4_ru_e = _split_w(W_ru_e, L, 2 * U)
    w0x_c_e, w0h_c_e, w4_c_e = _split_w(W_c_e, L, U)
    w0x_e = jnp.concatenate([_expand_ru(w0x_ru_e), w0x_c_e], axis=1)

    def _prep_d(w, dout):
        w0x, w0h, w4 = _split_w(w, 1, dout)
        wm = w.reshape(1 + U, NUM_MAT, dout)
        wmp = jnp.concatenate(
            [wm[1:], wm[:1], jnp.zeros((DIN_D - 1 - U, NUM_MAT, dout), F32)],
            axis=0)
        w0 = wmp[:, 0] - wmp[:, 2] - wmp[:, 4]
        w4d = jnp.concatenate(
            [wmp[:, 1], 2.0 * wmp[:, 2], wmp[:, 3], 2.0 * wmp[:, 4]], axis=0)
        return w0, w4d

    w0_ru_d, w4_ru_d = _prep_d(W_ru_d, 2 * U)
    w0_c_d, w4_c_d = _prep_d(W_c_d, U)

    out = pl.pallas_call(
        _fwd_kernel,
        out_shape=jax.ShapeDtypeStruct((HORIZON, B * N), F32),
    )(x_row, x_nd, support0.T.astype(BF16), support1.T.astype(BF16),
      w0x_e.astype(BF16),
      _expand_ru(w0h_ru_e).astype(BF16), _expand_ru(w4_ru_e).astype(BF16),
      _expand_ru(b_ru_e.reshape(1, 2 * U)),
      w0h_c_e.astype(BF16), w4_c_e.astype(BF16), b_c_e.reshape(1, U),
      _expand_ru(w0_ru_d).astype(BF16), _expand_ru(w4_ru_d).astype(BF16),
      _expand_ru(b_ru_d.reshape(1, 2 * U)),
      w0_c_d.astype(BF16), w4_c_d.astype(BF16), b_c_d.reshape(1, U),
      W_proj.T, b_proj.reshape(1, 1))

    return out.reshape(HORIZON, B, N)


# speedup vs baseline: 1.0387x; 1.0387x over previous
import jax
import jax.numpy as jnp
from jax.experimental import pallas as pl

N = 207
B = 16
L = 8
HORIZON = 8
U = 64
NUM_MAT = 5
DIN_D = 80
XP = 16
F32 = jnp.float32
BF16 = jnp.bfloat16

_DNT = (((0,), (0,)), ((), ()))


def _fwd_kernel(x_row_ref, x_nd_ref, s0t_ref, s1t_ref,
                w0x_e_ref, w0h_ru_e_ref, w4_ru_e_ref, b_ru_e_ref,
                w0h_c_e_ref, w4_c_e_ref, b_c_e_ref,
                w0_ru_d_ref, w4_ru_d_ref, b_ru_d_ref,
                w0_c_d_ref, w4_c_d_ref, b_c_d_ref,
                w_projt_ref, b_proj_ref,
                out_ref):
    s0t = s0t_ref[...]
    s1t = s1t_ref[...]

    def dotT(a, w):
        return jax.lax.dot_general(a, w, _DNT, preferred_element_type=F32)

    def diffuse(pieces, st):
        z1 = jnp.concatenate(
            [dotT(p, st).astype(BF16) for p in pieces], axis=0) \
            if len(pieces) > 1 else dotT(pieces[0], st).astype(BF16)
        z2 = jnp.dot(z1, st, preferred_element_type=F32).astype(BF16)
        return z1, z2

    w0x_e = w0x_e_ref[...]
    w0h_ru_e = w0h_ru_e_ref[...]
    w4_ru_e = w4_ru_e_ref[...]
    b_ru_e = b_ru_e_ref[...]
    w0h_c_e = w0h_c_e_ref[...]
    w4_c_e = w4_c_e_ref[...]
    b_c_e = b_c_e_ref[...]

    def ggate(state, zx, w0h, w4, b, g0x):
        sb = state.astype(BF16)
        s3 = sb.reshape(B, N, U)
        zh1a, zh2a = diffuse([s3[i] for i in range(B)], s0t)
        zh1b, zh2b = diffuse([s3[i] for i in range(B)], s1t)
        zx1a, zx2a, zx1b, zx2b = zx
        gates = []
        for i in range(B):
            hs = slice(i * U, (i + 1) * U)
            xs = slice(i * XP, (i + 1) * XP)
            zcat = jnp.concatenate(
                [zh1a[hs], zh2a[hs], zh1b[hs], zh2b[hs],
                 zx1a[xs], zx2a[xs], zx1b[xs], zx2b[xs]], axis=0)
            gates.append(dotT(zcat, w4))
        return (jnp.concatenate(gates, axis=0)
                + jnp.dot(sb, w0h, preferred_element_type=F32) + g0x + b)

    def enc_body(t, h):
        x_row = x_row_ref[pl.ds(t, 1)].reshape(B * N, L)
        x_nd = x_nd_ref[pl.ds(t, 1)].reshape(N, B * XP)
        zx1a, zx2a = diffuse([x_nd], s0t)
        zx1b, zx2b = diffuse([x_nd], s1t)
        zx = (zx1a, zx2a, zx1b, zx2b)
        g0x = jnp.dot(x_row, w0x_e, preferred_element_type=F32)
        ru_raw = ggate(h, zx, w0h_ru_e, w4_ru_e, b_ru_e, g0x[:, :2 * U])
        r = jax.nn.sigmoid(ru_raw[:, :U])
        u = jax.nn.sigmoid(ru_raw[:, U:])
        c = jnp.tanh(ggate(r * h, zx, w0h_c_e, w4_c_e, b_c_e, g0x[:, 2 * U:]))
        return u * h + (1.0 - u) * c

    h = jax.lax.fori_loop(0, L, enc_body, jnp.zeros((B * N, U), F32))

    w0_ru_d = w0_ru_d_ref[...]
    w4_ru_d = w4_ru_d_ref[...]
    b_ru_d = b_ru_d_ref[...]
    w0_c_d = w0_c_d_ref[...]
    w4_c_d = w4_c_d_ref[...]
    b_c_d = b_c_d_ref[...]
    w_projt = w_projt_ref[...]
    b_proj = b_proj_ref[0, 0]

    def gconv_d(cat, w0, w4, b):
        cat3 = cat.reshape(B, N, DIN_D)
        z1a, z2a = diffuse([cat3[i] for i in range(B)], s0t)
        z1b, z2b = diffuse([cat3[i] for i in range(B)], s1t)
        gates = []
        for i in range(B):
            sl = slice(i * DIN_D, (i + 1) * DIN_D)
            zcat = jnp.concatenate([z1a[sl], z2a[sl], z1b[sl], z2b[sl]], axis=0)
            gates.append(dotT(zcat, w4))
        return (jnp.concatenate(gates, axis=0)
                + jnp.dot(cat, w0, preferred_element_type=F32) + b)

    def dec_body(t, h):
        projt = jnp.dot(w_projt, h.T, preferred_element_type=F32) + b_proj
        xin = jnp.where(t == 0, jnp.zeros_like(projt), projt).T
        xin16 = jnp.pad(xin, ((0, 0), (0, XP - 1)))
        cat = jnp.concatenate([h, xin16], axis=1).astype(BF16)
        ru_raw = gconv_d(cat, w0_ru_d, w4_ru_d, b_ru_d)
        r = jax.nn.sigmoid(ru_raw[:, :U])
        u = jax.nn.sigmoid(ru_raw[:, U:])
        cat2 = jnp.concatenate([r * h, xin16], axis=1).astype(BF16)
        c = jnp.tanh(gconv_d(cat2, w0_c_d, w4_c_d, b_c_d))
        h2 = u * h + (1.0 - u) * c
        proj2 = jnp.dot(w_projt, h2.T, preferred_element_type=F32) + b_proj
        out_ref[pl.ds(t, 1)] = proj2
        return h2

    jax.lax.fori_loop(0, HORIZON, dec_body, h)


def _expand_ru(w):
    return w


def _split_w(w, dx, dout):
    din = dx + U
    wm = w.reshape(din, NUM_MAT, dout)
    wx, wh = wm[:dx], wm[dx:]
    w0x = wx[:, 0] - wx[:, 2] - wx[:, 4]
    w0h = wh[:, 0] - wh[:, 2] - wh[:, 4]
    pad = jnp.zeros((XP - dx, dout), F32)
    w4 = jnp.concatenate(
        [wh[:, 1], 2.0 * wh[:, 2], wh[:, 3], 2.0 * wh[:, 4],
         wx[:, 1], pad, 2.0 * wx[:, 2], pad, wx[:, 3], pad,
         2.0 * wx[:, 4], pad], axis=0)
    return w0x, w0h, w4


def kernel(inputs, support0, support1, W_ru_e, b_ru_e, W_c_e, b_c_e,
           W_ru_d, b_ru_d, W_c_d, b_c_d, W_proj, b_proj):
    x_row = inputs.reshape(L, B * N, L).astype(BF16)
    x_nd = jnp.pad(inputs.reshape(L, B, N, L).transpose(0, 2, 1, 3),
                   ((0, 0), (0, 0), (0, 0), (0, XP - L))
                   ).reshape(L, N, B * XP).astype(BF16)

    w0x_ru_e, w0h_ru_e, w4_ru_e = _split_w(W_ru_e, L, 2 * U)
    w0x_c_e, w0h_c_e, w4_c_e = _split_w(W_c_e, L, U)
    w0x_e = jnp.concatenate([_expand_ru(w0x_ru_e), w0x_c_e], axis=1)

    def _prep_d(w, dout):
        w0x, w0h, w4 = _split_w(w, 1, dout)
        wm = w.reshape(1 + U, NUM_MAT, dout)
        wmp = jnp.concatenate(
            [wm[1:], wm[:1], jnp.zeros((DIN_D - 1 - U, NUM_MAT, dout), F32)],
            axis=0)
        w0 = wmp[:, 0] - wmp[:, 2] - wmp[:, 4]
        w4d = jnp.concatenate(
            [wmp[:, 1], 2.0 * wmp[:, 2], wmp[:, 3], 2.0 * wmp[:, 4]], axis=0)
        return w0, w4d

    w0_ru_d, w4_ru_d = _prep_d(W_ru_d, 2 * U)
    w0_c_d, w4_c_d = _prep_d(W_c_d, U)

    out = pl.pallas_call(
        _fwd_kernel,
        out_shape=jax.ShapeDtypeStruct((HORIZON, B * N), F32),
    )(x_row, x_nd, support0.T.astype(BF16), support1.T.astype(BF16),
      w0x_e.astype(BF16),
      _expand_ru(w0h_ru_e).astype(BF16), _expand_ru(w4_ru_e).astype(BF16),
      _expand_ru(b_ru_e.reshape(1, 2 * U)),
      w0h_c_e.astype(BF16), w4_c_e.astype(BF16), b_c_e.reshape(1, U),
      _expand_ru(w0_ru_d).astype(BF16), _expand_ru(w4_ru_d).astype(BF16),
      _expand_ru(b_ru_d.reshape(1, 2 * U)),
      w0_c_d.astype(BF16), w4_c_d.astype(BF16), b_c_d.reshape(1, U),
      W_proj.T, b_proj.reshape(1, 1))

    return out.reshape(HORIZON, B, N)


# speedup vs baseline: 1.0993x; 1.0583x over previous
import jax
import jax.numpy as jnp
from jax.experimental import pallas as pl

N = 207
B = 16
L = 8
HORIZON = 8
U = 64
NUM_MAT = 5
DIN = 80
F32 = jnp.float32
BF16 = jnp.bfloat16

_DNT = (((0,), (0,)), ((), ()))


def _fwd_kernel(x_all_ref, s0t_ref, s1t_ref,
                w0_ru_e_ref, w4_ru_e_ref, b_ru_e_ref,
                w0_c_e_ref, w4_c_e_ref, b_c_e_ref,
                w0_ru_d_ref, w4_ru_d_ref, b_ru_d_ref,
                w0_c_d_ref, w4_c_d_ref, b_c_d_ref,
                w_projt_ref, b_proj_ref,
                out_ref):
    s0t = s0t_ref[...]
    s1t = s1t_ref[...]

    def dotT(a, w):
        return jax.lax.dot_general(a, w, _DNT, preferred_element_type=F32)

    def gconv(cat, w0, w4, b):
        g0 = jnp.dot(cat, w0, preferred_element_type=F32)
        cat3 = cat.reshape(B, N, DIN)
        z1a = jnp.concatenate(
            [dotT(cat3[i], s0t).astype(BF16) for i in range(B)], axis=0)
        z1b = jnp.concatenate(
            [dotT(cat3[i], s1t).astype(BF16) for i in range(B)], axis=0)
        z2a = jnp.dot(z1a, s0t, preferred_element_type=F32).astype(BF16)
        z2b = jnp.dot(z1b, s1t, preferred_element_type=F32).astype(BF16)
        gates = []
        for i in range(B):
            sl = slice(i * DIN, (i + 1) * DIN)
            zcat = jnp.concatenate([z1a[sl], z2a[sl], z1b[sl], z2b[sl]], axis=0)
            gates.append(dotT(zcat, w4))
        return g0 + jnp.concatenate(gates, axis=0) + b

    def cell(x16, h, w0_ru, w4_ru, b_ru, w0_c, w4_c, b_c):
        cat = jnp.concatenate([h, x16], axis=1).astype(BF16)
        ru = jax.nn.sigmoid(gconv(cat, w0_ru, w4_ru, b_ru))
        r = ru[:, :U]
        u = ru[:, U:]
        cat2 = jnp.concatenate([r * h, x16], axis=1).astype(BF16)
        c = jnp.tanh(gconv(cat2, w0_c, w4_c, b_c))
        return u * h + (1.0 - u) * c

    w0_ru_e = w0_ru_e_ref[...]
    w4_ru_e = w4_ru_e_ref[...]
    b_ru_e = b_ru_e_ref[...]
    w0_c_e = w0_c_e_ref[...]
    w4_c_e = w4_c_e_ref[...]
    b_c_e = b_c_e_ref[...]

    def enc_body(t, h):
        x16 = x_all_ref[pl.ds(t, 1)].reshape(B * N, DIN - U)
        return cell(x16, h, w0_ru_e, w4_ru_e, b_ru_e, w0_c_e, w4_c_e, b_c_e)

    h = jax.lax.fori_loop(0, L, enc_body, jnp.zeros((B * N, U), F32))

    w0_ru_d = w0_ru_d_ref[...]
    w4_ru_d = w4_ru_d_ref[...]
    b_ru_d = b_ru_d_ref[...]
    w0_c_d = w0_c_d_ref[...]
    w4_c_d = w4_c_d_ref[...]
    b_c_d = b_c_d_ref[...]
    w_projt = w_projt_ref[...]
    b_proj = b_proj_ref[0, 0]

    def dec_body(t, h):
        projt = jnp.dot(w_projt, h.T, preferred_element_type=F32) + b_proj
        xin = jnp.where(t == 0, jnp.zeros_like(projt), projt).T
        x16 = jnp.pad(xin, ((0, 0), (0, DIN - U - 1)))
        h2 = cell(x16, h, w0_ru_d, w4_ru_d, b_ru_d, w0_c_d, w4_c_d, b_c_d)
        proj2 = jnp.dot(w_projt, h2.T, preferred_element_type=F32) + b_proj
        out_ref[pl.ds(t, 1)] = proj2
        return h2

    jax.lax.fori_loop(0, HORIZON, dec_body, h)


def _prep_w(w, dx, dout):
    din = dx + U
    wm = w.reshape(din, NUM_MAT, dout)
    wm = jnp.concatenate(
        [wm[dx:], wm[:dx], jnp.zeros((DIN - din, NUM_MAT, dout), F32)],
        axis=0)
    w0 = wm[:, 0] - wm[:, 2] - wm[:, 4]
    w4 = jnp.concatenate(
        [wm[:, 1], 2.0 * wm[:, 2], wm[:, 3], 2.0 * wm[:, 4]], axis=0)
    return w0.astype(BF16), w4.astype(BF16)


def kernel(inputs, support0, support1, W_ru_e, b_ru_e, W_c_e, b_c_e,
           W_ru_d, b_ru_d, W_c_d, b_c_d, W_proj, b_proj):
    x_all = jnp.pad(inputs.reshape(L, B * N, L),
                    ((0, 0), (0, 0), (0, DIN - U - L))).astype(BF16)

    w0_ru_e, w4_ru_e = _prep_w(W_ru_e, L, 2 * U)
    w0_c_e, w4_c_e = _prep_w(W_c_e, L, U)
    w0_ru_d, w4_ru_d = _prep_w(W_ru_d, 1, 2 * U)
    w0_c_d, w4_c_d = _prep_w(W_c_d, 1, U)

    out = pl.pallas_call(
        _fwd_kernel,
        out_shape=jax.ShapeDtypeStruct((HORIZON, B * N), F32),
    )(x_all, support0.T.astype(BF16), support1.T.astype(BF16),
      w0_ru_e, w4_ru_e, b_ru_e.reshape(1, 2 * U),
      w0_c_e, w4_c_e, b_c_e.reshape(1, U),
      w0_ru_d, w4_ru_d, b_ru_d.reshape(1, 2 * U),
      w0_c_d, w4_c_d, b_c_d.reshape(1, U),
      W_proj.T, b_proj.reshape(1, 1))

    return out.reshape(HORIZON, B, N)
